# baseline (speedup 1.0000x reference)
import functools

import jax
import jax.numpy as jnp
from jax import lax
from jax.experimental import pallas as pl
from jax.experimental.pallas import tpu as pltpu
from jax.experimental.pallas import tpu_sc as plsc

N = 10000
NPAD = 10240
E = 320000
NC = 2
NS = 16
NW = NC * NS
CH = 128
EROWS = E // CH
RPW = EROWS // NW
XTRA = EROWS - RPW * NW
CPW = RPW + 1
TRASH = N
ROWS_PER_TILE = NPAD // NS
RPT = EROWS // NS
XT = EROWS - RPT * NS
CPT = RPT + 1
BLK = 2048
GRID = NPAD // BLK


def _make_agg(D, gather=True):
  mesh = plsc.VectorSubcoreMesh(core_axis_name="c", subcore_axis_name="s",
                                num_cores=NC, num_subcores=NS)

  def body(table, edges, zeros, out, src_v, dst_v, buf0, buf1, acc, tbl,
           gsem0, gsem1, ssem0, ssem1):
    c = lax.axis_index("c")
    s = lax.axis_index("s")
    w = c * NS + s
    row0 = w * RPW + jnp.minimum(w, XTRA)

    for k in range(CH // 16):
      src_v[RPW, pl.ds(k * 16, 16)] = jnp.zeros((16,), jnp.int32)
      dst_v[RPW, pl.ds(k * 16, 16)] = jnp.full((16,), TRASH, jnp.int32)
    if gather:
      pltpu.sync_copy(edges.at[0, pl.ds(row0, RPW)], src_v.at[pl.ds(0, RPW)])
    pltpu.sync_copy(edges.at[1, pl.ds(row0, RPW)], dst_v.at[pl.ds(0, RPW)])

    @pl.when(w < XTRA)
    def _():
      if gather:
        pltpu.sync_copy(edges.at[0, pl.ds(row0 + RPW, 1)],
                        src_v.at[pl.ds(RPW, 1)])
      pltpu.sync_copy(edges.at[1, pl.ds(row0 + RPW, 1)],
                      dst_v.at[pl.ds(RPW, 1)])

    pltpu.sync_copy(zeros, acc.at[pl.ds(s * ROWS_PER_TILE, ROWS_PER_TILE)])
    if gather:
      pltpu.sync_copy(table.at[pl.ds(s * ROWS_PER_TILE, ROWS_PER_TILE)],
                      tbl.at[pl.ds(s * ROWS_PER_TILE, ROWS_PER_TILE)])
    plsc.subcore_barrier()

    if gather:
      pltpu.async_copy(tbl.at[src_v.at[0]], buf0, gsem0)

      def step(j, carry):
        i0 = 2 * j
        i1 = 2 * j + 1
        pltpu.make_async_copy(tbl.at[src_v.at[i0]], buf0, gsem0).wait()
        pltpu.async_copy(tbl.at[src_v.at[i1]], buf1, gsem1)
        pltpu.sync_copy(buf0, acc.at[dst_v.at[i0]], add=True)
        pltpu.make_async_copy(tbl.at[src_v.at[i1]], buf1, gsem1).wait()
        pltpu.async_copy(tbl.at[src_v.at[i0 + 2]], buf0, gsem0)
        pltpu.sync_copy(buf1, acc.at[dst_v.at[i1]], add=True)
        return carry

      lax.fori_loop(0, CPW // 2, step, 0)
      pltpu.make_async_copy(tbl.at[src_v.at[CPW - 1]], buf0, gsem0).wait()
      pltpu.sync_copy(buf0, acc.at[dst_v.at[CPW - 1]], add=True)
    else:
      pltpu.sync_copy(table, buf0)

      def step(j, carry):
        c0 = pltpu.async_copy(buf0, acc.at[dst_v.at[2 * j]], ssem0, add=True)
        c1 = pltpu.async_copy(buf0, acc.at[dst_v.at[2 * j + 1]], ssem1,
                              add=True)
        c0.wait()
        c1.wait()
        return carry

      lax.fori_loop(0, CPW // 2, step, 0)
      pltpu.sync_copy(buf0, acc.at[dst_v.at[CPW - 1]], add=True)

    plsc.subcore_barrier()
    pltpu.sync_copy(acc.at[pl.ds(s * ROWS_PER_TILE, ROWS_PER_TILE)],
                    out.at[pl.ds(s * ROWS_PER_TILE, ROWS_PER_TILE),
                           pl.ds(c * D, D)])

  return pl.kernel(
      body,
      out_type=jax.ShapeDtypeStruct((NPAD, NC * D), jnp.float32),
      mesh=mesh,
      scratch_types=[
          pltpu.VMEM((CPW, CH), jnp.int32),
          pltpu.VMEM((CPW, CH), jnp.int32),
          pltpu.VMEM((CH, D), jnp.float32),
          pltpu.VMEM((CH, D), jnp.float32),
          pltpu.VMEM_SHARED((NPAD, D), jnp.float32),
          pltpu.VMEM_SHARED((NPAD, D) if gather else (CH, D), jnp.float32),
          pltpu.SemaphoreType.DMA,
          pltpu.SemaphoreType.DMA,
          pltpu.SemaphoreType.DMA,
          pltpu.SemaphoreType.DMA,
      ],
      compiler_params=pltpu.CompilerParams(use_tc_tiling_on_sc=False),
  )


def _make_agg_fs(DH):
  mesh = plsc.VectorSubcoreMesh(core_axis_name="c", subcore_axis_name="s",
                                num_cores=NC, num_subcores=NS)

  def body(table, edges, zeros, out, src_v, dst_v, buf0, buf1, acc, tbl,
           gsem0, gsem1):
    c = lax.axis_index("c")
    s = lax.axis_index("s")
    row0 = s * RPT + jnp.minimum(s, XT)

    for k in range(CH // 16):
      src_v[RPT, pl.ds(k * 16, 16)] = jnp.zeros((16,), jnp.int32)
      dst_v[RPT, pl.ds(k * 16, 16)] = jnp.full((16,), TRASH, jnp.int32)
    pltpu.sync_copy(edges.at[0, pl.ds(row0, RPT)], src_v.at[pl.ds(0, RPT)])
    pltpu.sync_copy(edges.at[1, pl.ds(row0, RPT)], dst_v.at[pl.ds(0, RPT)])

    @pl.when(s < XT)
    def _():
      pltpu.sync_copy(edges.at[0, pl.ds(row0 + RPT, 1)],
                      src_v.at[pl.ds(RPT, 1)])
      pltpu.sync_copy(edges.at[1, pl.ds(row0 + RPT, 1)],
                      dst_v.at[pl.ds(RPT, 1)])

    pltpu.sync_copy(zeros, acc.at[pl.ds(s * ROWS_PER_TILE, ROWS_PER_TILE)])
    pltpu.sync_copy(
        table.at[pl.ds(s * ROWS_PER_TILE, ROWS_PER_TILE), pl.ds(c * DH, DH)],
        tbl.at[pl.ds(s * ROWS_PER_TILE, ROWS_PER_TILE)])
    plsc.subcore_barrier()

    pltpu.async_copy(tbl.at[src_v.at[0]], buf0, gsem0)

    def step(j, carry):
      i0 = 2 * j
      i1 = 2 * j + 1
      pltpu.make_async_copy(tbl.at[src_v.at[i0]], buf0, gsem0).wait()
      pltpu.async_copy(tbl.at[src_v.at[i1]], buf1, gsem1)
      pltpu.sync_copy(buf0, acc.at[dst_v.at[i0]], add=True)
      pltpu.make_async_copy(tbl.at[src_v.at[i1]], buf1, gsem1).wait()
      pltpu.async_copy(tbl.at[src_v.at[i0 + 2]], buf0, gsem0)
      pltpu.sync_copy(buf1, acc.at[dst_v.at[i1]], add=True)
      return carry

    lax.fori_loop(0, CPT // 2, step, 0)
    pltpu.make_async_copy(tbl.at[src_v.at[CPT - 1]], buf0, gsem0).wait()
    pltpu.sync_copy(buf0, acc.at[dst_v.at[CPT - 1]], add=True)

    plsc.subcore_barrier()
    pltpu.sync_copy(
        acc.at[pl.ds(s * ROWS_PER_TILE, ROWS_PER_TILE)],
        out.at[pl.ds(s * ROWS_PER_TILE, ROWS_PER_TILE), pl.ds(c * DH, DH)])

  return pl.kernel(
      body,
      out_type=jax.ShapeDtypeStruct((NPAD, 2 * DH), jnp.float32),
      mesh=mesh,
      scratch_types=[
          pltpu.VMEM((CPT, CH), jnp.int32),
          pltpu.VMEM((CPT, CH), jnp.int32),
          pltpu.VMEM((CH, DH), jnp.float32),
          pltpu.VMEM((CH, DH), jnp.float32),
          pltpu.VMEM_SHARED((NPAD, DH), jnp.float32),
          pltpu.VMEM_SHARED((NPAD, DH), jnp.float32),
          pltpu.SemaphoreType.DMA,
          pltpu.SemaphoreType.DMA,
      ],
      compiler_params=pltpu.CompilerParams(use_tc_tiling_on_sc=False),
  )


def _mm_body(x_ref, w_ref, o_ref):
  o_ref[...] = jnp.dot(x_ref[...], w_ref[...],
                       preferred_element_type=jnp.float32)


def _scale_body(m_ref, deg_ref, o_ref, d_ref):
  dis = lax.rsqrt(1.0 + deg_ref[:, 0:1] + deg_ref[:, 16:17])
  o_ref[...] = dis * m_ref[...]
  d_ref[...] = jnp.broadcast_to(dis, (BLK, 8))


def _mid_body(agg_ref, p_ref, b_ref, w_ref, dis_ref, o_ref):
  dis = dis_ref[:, 0:1]
  t = agg_ref[...] + p_ref[...]
  t = jnp.maximum(dis * t + b_ref[0:1, :], 0.0)
  o_ref[...] = dis * jnp.dot(t, w_ref[...],
                             preferred_element_type=jnp.float32)


def _last_body(parts_ref, p_ref, b_ref, w_ref, bl_ref, dis_ref, o_ref):
  dis = dis_ref[:, 0:1]
  d = p_ref.shape[1]
  t = parts_ref[:, 0:d] + parts_ref[:, d:2 * d] + p_ref[...]
  t = jnp.maximum(dis * t + b_ref[0:1, :], 0.0)
  o_ref[...] = jax.nn.sigmoid(
      jnp.dot(t, w_ref[...], preferred_element_type=jnp.float32)
      + bl_ref[0:1, 0:1])


def _row_spec(d):
  return pl.BlockSpec((BLK, d), lambda i: (i, 0))


def _parts_spec(d):
  return pl.BlockSpec((NC, BLK, d), lambda i: (0, i, 0))


def _full_spec(a, b):
  return pl.BlockSpec((a, b), lambda i: (0, 0))


def _pk_spec(k):
  return pl.BlockSpec((BLK // k, 128), lambda i: (i, 0))


def _tc_mm(x, w):
  dout = w.shape[1]
  return pl.pallas_call(
      _mm_body,
      grid=(GRID,),
      in_specs=[_row_spec(x.shape[1]), _full_spec(*w.shape)],
      out_specs=_row_spec(dout),
      out_shape=jax.ShapeDtypeStruct((NPAD, dout), jnp.float32),
  )(x, w)


def _tc_scale(m, degp):
  d = m.shape[1]
  return pl.pallas_call(
      _scale_body,
      grid=(GRID,),
      in_specs=[_row_spec(d), _row_spec(32)],
      out_specs=[_row_spec(d), _row_spec(8)],
      out_shape=[jax.ShapeDtypeStruct((NPAD, d), jnp.float32),
                 jax.ShapeDtypeStruct((NPAD, 8), jnp.float32)],
  )(m, degp)


def _tc_mid(agg, p, b8, w, dis8):
  d = w.shape[0]
  dout = w.shape[1]
  return pl.pallas_call(
      _mid_body,
      grid=(GRID,),
      in_specs=[_row_spec(d), _row_spec(d), _full_spec(8, d),
                _full_spec(*w.shape), _row_spec(8)],
      out_specs=_row_spec(dout),
      out_shape=jax.ShapeDtypeStruct((NPAD, dout), jnp.float32),
  )(agg, p, b8, w, dis8)


def _tc_last(parts, p, b8, wl, bl8, dis8):
  d = wl.shape[0]
  return pl.pallas_call(
      _last_body,
      grid=(GRID,),
      in_specs=[_row_spec(2 * d), _row_spec(d), _full_spec(8, d),
                _full_spec(*wl.shape), _full_spec(8, 8), _row_spec(8)],
      out_specs=_row_spec(1),
      out_shape=jax.ShapeDtypeStruct((N, 1), jnp.float32),
  )(parts, p, b8, wl, bl8, dis8)


@jax.jit
def kernel(x, edge_index, W1, b1, W2, b2, W3, b3, Wl, bl):
  e3 = edge_index.reshape(2, EROWS, CH)

  b18 = jnp.broadcast_to(b1, (8, b1.shape[0]))
  b28 = jnp.broadcast_to(b2, (8, b2.shape[0]))
  b38 = jnp.broadcast_to(b3, (8, b3.shape[0]))
  bl8 = jnp.broadcast_to(bl, (8, 8))

  z16 = jnp.zeros((ROWS_PER_TILE, 16), jnp.float32)
  z32 = jnp.zeros((ROWS_PER_TILE, 32), jnp.float32)
  ones16 = jnp.ones((CH, 16), jnp.float32)

  degp = _make_agg(16, gather=False)(ones16, e3, z16)
  m1 = _tc_mm(x, W1)
  p1, dis8 = _tc_scale(m1, degp)
  a1 = _make_agg_fs(32)(p1, e3, z32)
  p2 = _tc_mid(a1, p1, b18, W2, dis8)
  a2 = _make_agg_fs(16)(p2, e3, z16)
  p3 = _tc_mid(a2, p2, b28, W3, dis8)
  a3 = _make_agg(16)(p3, e3, z16)
  return _tc_last(a3, p3, b38, Wl, bl8, dis8)


# speedup vs baseline: 1.0013x; 1.0013x over previous
import jax
import jax.numpy as jnp
from jax import lax
from jax.experimental import pallas as pl
from jax.experimental.pallas import tpu as pltpu
from jax.experimental.pallas import tpu_sc as plsc

N = 10000
NPAD = 10240
E = 320000
NC = 2
NS = 16
NW = NC * NS
CH = 128
EROWS = E // CH
RPW = EROWS // NW
XTRA = EROWS - RPW * NW
CPW = RPW + 1
TRASH = N
ROWS_PER_TILE = NPAD // NS
RPT = EROWS // NS
XT = EROWS - RPT * NS
CPT = RPT + 1
BLK = 2048
GRID = NPAD // BLK


def _make_agg(D, gather=True):
  mesh = plsc.VectorSubcoreMesh(core_axis_name="c", subcore_axis_name="s",
                                num_cores=NC, num_subcores=NS)

  def body(table, edges, zeros, out, src_v, dst_v, buf0, buf1, acc, tbl,
           gsem0, gsem1, ssem0, ssem1):
    c = lax.axis_index("c")
    s = lax.axis_index("s")
    w = c * NS + s
    row0 = w * RPW + jnp.minimum(w, XTRA)

    for k in range(CH // 16):
      src_v[RPW, pl.ds(k * 16, 16)] = jnp.zeros((16,), jnp.int32)
      dst_v[RPW, pl.ds(k * 16, 16)] = jnp.full((16,), TRASH, jnp.int32)
    if gather:
      pltpu.sync_copy(edges.at[0, pl.ds(row0, RPW)], src_v.at[pl.ds(0, RPW)])
    pltpu.sync_copy(edges.at[1, pl.ds(row0, RPW)], dst_v.at[pl.ds(0, RPW)])

    @pl.when(w < XTRA)
    def _():
      if gather:
        pltpu.sync_copy(edges.at[0, pl.ds(row0 + RPW, 1)],
                        src_v.at[pl.ds(RPW, 1)])
      pltpu.sync_copy(edges.at[1, pl.ds(row0 + RPW, 1)],
                      dst_v.at[pl.ds(RPW, 1)])

    pltpu.sync_copy(zeros, acc.at[pl.ds(s * ROWS_PER_TILE, ROWS_PER_TILE)])
    if gather:
      pltpu.sync_copy(table.at[pl.ds(s * ROWS_PER_TILE, ROWS_PER_TILE)],
                      tbl.at[pl.ds(s * ROWS_PER_TILE, ROWS_PER_TILE)])
    plsc.subcore_barrier()

    if gather:
      pltpu.async_copy(tbl.at[src_v.at[0]], buf0, gsem0)

      def step(j, carry):
        i0 = 2 * j
        i1 = 2 * j + 1
        pltpu.make_async_copy(tbl.at[src_v.at[i0]], buf0, gsem0).wait()
        pltpu.async_copy(tbl.at[src_v.at[i1]], buf1, gsem1)
        pltpu.sync_copy(buf0, acc.at[dst_v.at[i0]], add=True)
        pltpu.make_async_copy(tbl.at[src_v.at[i1]], buf1, gsem1).wait()
        pltpu.async_copy(tbl.at[src_v.at[i0 + 2]], buf0, gsem0)
        pltpu.sync_copy(buf1, acc.at[dst_v.at[i1]], add=True)
        return carry

      lax.fori_loop(0, CPW // 2, step, 0)
      pltpu.make_async_copy(tbl.at[src_v.at[CPW - 1]], buf0, gsem0).wait()
      pltpu.sync_copy(buf0, acc.at[dst_v.at[CPW - 1]], add=True)
    else:
      pltpu.sync_copy(table, buf0)

      def step(j, carry):
        c0 = pltpu.async_copy(buf0, acc.at[dst_v.at[2 * j]], ssem0, add=True)
        c1 = pltpu.async_copy(buf0, acc.at[dst_v.at[2 * j + 1]], ssem1,
                              add=True)
        c0.wait()
        c1.wait()
        return carry

      lax.fori_loop(0, CPW // 2, step, 0)
      pltpu.sync_copy(buf0, acc.at[dst_v.at[CPW - 1]], add=True)

    plsc.subcore_barrier()
    pltpu.sync_copy(acc.at[pl.ds(s * ROWS_PER_TILE, ROWS_PER_TILE)],
                    out.at[pl.ds(s * ROWS_PER_TILE, ROWS_PER_TILE),
                           pl.ds(c * D, D)])

  return pl.kernel(
      body,
      out_type=jax.ShapeDtypeStruct((NPAD, NC * D), jnp.float32),
      mesh=mesh,
      scratch_types=[
          pltpu.VMEM((CPW, CH), jnp.int32),
          pltpu.VMEM((CPW, CH), jnp.int32),
          pltpu.VMEM((CH, D), jnp.float32),
          pltpu.VMEM((CH, D), jnp.float32),
          pltpu.VMEM_SHARED((NPAD, D), jnp.float32),
          pltpu.VMEM_SHARED((NPAD, D) if gather else (CH, D), jnp.float32),
          pltpu.SemaphoreType.DMA,
          pltpu.SemaphoreType.DMA,
          pltpu.SemaphoreType.DMA,
          pltpu.SemaphoreType.DMA,
      ],
      compiler_params=pltpu.CompilerParams(use_tc_tiling_on_sc=False),
  )


def _make_agg_fs(DH):
  mesh = plsc.VectorSubcoreMesh(core_axis_name="c", subcore_axis_name="s",
                                num_cores=NC, num_subcores=NS)

  def body(table, edges, zeros, out, src_v, dst_v, buf0, buf1, acc, tbl,
           gsem0, gsem1):
    c = lax.axis_index("c")
    s = lax.axis_index("s")
    row0 = s * RPT + jnp.minimum(s, XT)

    for k in range(CH // 16):
      src_v[RPT, pl.ds(k * 16, 16)] = jnp.zeros((16,), jnp.int32)
      dst_v[RPT, pl.ds(k * 16, 16)] = jnp.full((16,), TRASH, jnp.int32)
    pltpu.sync_copy(edges.at[0, pl.ds(row0, RPT)], src_v.at[pl.ds(0, RPT)])
    pltpu.sync_copy(edges.at[1, pl.ds(row0, RPT)], dst_v.at[pl.ds(0, RPT)])

    @pl.when(s < XT)
    def _():
      pltpu.sync_copy(edges.at[0, pl.ds(row0 + RPT, 1)],
                      src_v.at[pl.ds(RPT, 1)])
      pltpu.sync_copy(edges.at[1, pl.ds(row0 + RPT, 1)],
                      dst_v.at[pl.ds(RPT, 1)])

    pltpu.sync_copy(zeros, acc.at[pl.ds(s * ROWS_PER_TILE, ROWS_PER_TILE)])
    pltpu.sync_copy(
        table.at[pl.ds(s * ROWS_PER_TILE, ROWS_PER_TILE), pl.ds(c * DH, DH)],
        tbl.at[pl.ds(s * ROWS_PER_TILE, ROWS_PER_TILE)])
    plsc.subcore_barrier()

    pltpu.async_copy(tbl.at[src_v.at[0]], buf0, gsem0)

    def step(j, carry):
      i0 = 2 * j
      i1 = 2 * j + 1
      pltpu.make_async_copy(tbl.at[src_v.at[i0]], buf0, gsem0).wait()
      pltpu.async_copy(tbl.at[src_v.at[i1]], buf1, gsem1)
      pltpu.sync_copy(buf0, acc.at[dst_v.at[i0]], add=True)
      pltpu.make_async_copy(tbl.at[src_v.at[i1]], buf1, gsem1).wait()
      pltpu.async_copy(tbl.at[src_v.at[i0 + 2]], buf0, gsem0)
      pltpu.sync_copy(buf1, acc.at[dst_v.at[i1]], add=True)
      return carry

    lax.fori_loop(0, CPT // 2, step, 0)
    pltpu.make_async_copy(tbl.at[src_v.at[CPT - 1]], buf0, gsem0).wait()
    pltpu.sync_copy(buf0, acc.at[dst_v.at[CPT - 1]], add=True)

    plsc.subcore_barrier()
    pltpu.sync_copy(
        acc.at[pl.ds(s * ROWS_PER_TILE, ROWS_PER_TILE)],
        out.at[pl.ds(s * ROWS_PER_TILE, ROWS_PER_TILE), pl.ds(c * DH, DH)])

  return pl.kernel(
      body,
      out_type=jax.ShapeDtypeStruct((NPAD, 2 * DH), jnp.float32),
      mesh=mesh,
      scratch_types=[
          pltpu.VMEM((CPT, CH), jnp.int32),
          pltpu.VMEM((CPT, CH), jnp.int32),
          pltpu.VMEM((CH, DH), jnp.float32),
          pltpu.VMEM((CH, DH), jnp.float32),
          pltpu.VMEM_SHARED((NPAD, DH), jnp.float32),
          pltpu.VMEM_SHARED((NPAD, DH), jnp.float32),
          pltpu.SemaphoreType.DMA,
          pltpu.SemaphoreType.DMA,
      ],
      compiler_params=pltpu.CompilerParams(use_tc_tiling_on_sc=False),
  )


def _mm_body(x_ref, w_ref, o_ref):
  o_ref[...] = jnp.dot(x_ref[...], w_ref[...],
                       preferred_element_type=jnp.float32)


def _scale_body(m_ref, deg_ref, o_ref, d_ref):
  dis = lax.rsqrt(1.0 + deg_ref[:, 0:1] + deg_ref[:, 16:17])
  o_ref[...] = dis * m_ref[...]
  d_ref[...] = jnp.broadcast_to(dis, (BLK, 8))


def _mid_body(agg_ref, p_ref, b_ref, w_ref, dis_ref, o_ref):
  dis = dis_ref[:, 0:1]
  t = agg_ref[...] + p_ref[...]
  t = jnp.maximum(dis * t + b_ref[0:1, :], 0.0)
  o_ref[...] = dis * jnp.dot(t, w_ref[...],
                             preferred_element_type=jnp.float32)


def _last_body(parts_ref, p_ref, b_ref, w_ref, bl_ref, dis_ref, o_ref):
  dis = dis_ref[:, 0:1]
  d = p_ref.shape[1]
  t = parts_ref[:, 0:d] + parts_ref[:, d:2 * d] + p_ref[...]
  t = jnp.maximum(dis * t + b_ref[0:1, :], 0.0)
  o_ref[...] = jax.nn.sigmoid(
      jnp.dot(t, w_ref[...], preferred_element_type=jnp.float32)
      + bl_ref[0:1, 0:1])


def _row_spec(d):
  return pl.BlockSpec((BLK, d), lambda i: (i, 0))


def _full_spec(a, b):
  return pl.BlockSpec((a, b), lambda i: (0, 0))


def _tc_mm(x, w):
  dout = w.shape[1]
  return pl.pallas_call(
      _mm_body,
      grid=(GRID,),
      in_specs=[_row_spec(x.shape[1]), _full_spec(*w.shape)],
      out_specs=_row_spec(dout),
      out_shape=jax.ShapeDtypeStruct((NPAD, dout), jnp.float32),
  )(x, w)


def _tc_scale(m, degp):
  d = m.shape[1]
  return pl.pallas_call(
      _scale_body,
      grid=(GRID,),
      in_specs=[_row_spec(d), _row_spec(32)],
      out_specs=[_row_spec(d), _row_spec(8)],
      out_shape=[jax.ShapeDtypeStruct((NPAD, d), jnp.float32),
                 jax.ShapeDtypeStruct((NPAD, 8), jnp.float32)],
  )(m, degp)


def _tc_mid(agg, p, b8, w, dis8):
  d = w.shape[0]
  dout = w.shape[1]
  return pl.pallas_call(
      _mid_body,
      grid=(GRID,),
      in_specs=[_row_spec(d), _row_spec(d), _full_spec(8, d),
                _full_spec(*w.shape), _row_spec(8)],
      out_specs=_row_spec(dout),
      out_shape=jax.ShapeDtypeStruct((NPAD, dout), jnp.float32),
  )(agg, p, b8, w, dis8)


def _tc_last(parts, p, b8, wl, bl8, dis8):
  d = wl.shape[0]
  return pl.pallas_call(
      _last_body,
      grid=(GRID,),
      in_specs=[_row_spec(2 * d), _row_spec(d), _full_spec(8, d),
                _full_spec(*wl.shape), _full_spec(8, 8), _row_spec(8)],
      out_specs=_row_spec(1),
      out_shape=jax.ShapeDtypeStruct((N, 1), jnp.float32),
  )(parts, p, b8, wl, bl8, dis8)


@jax.jit
def kernel(x, edge_index, W1, b1, W2, b2, W3, b3, Wl, bl):
  e3 = edge_index.reshape(2, EROWS, CH)

  b18 = jnp.broadcast_to(b1, (8, b1.shape[0]))
  b28 = jnp.broadcast_to(b2, (8, b2.shape[0]))
  b38 = jnp.broadcast_to(b3, (8, b3.shape[0]))
  bl8 = jnp.broadcast_to(bl, (8, 8))

  z16 = jnp.zeros((ROWS_PER_TILE, 16), jnp.float32)
  z32 = jnp.zeros((ROWS_PER_TILE, 32), jnp.float32)
  ones16 = jnp.ones((CH, 16), jnp.float32)

  degp = _make_agg(16, gather=False)(ones16, e3, z16)
  m1 = _tc_mm(x, W1)
  p1, dis8 = _tc_scale(m1, degp)
  a1 = _make_agg_fs(32)(p1, e3, z32)
  p2 = _tc_mid(a1, p1, b18, W2, dis8)
  a2 = _make_agg_fs(16)(p2, e3, z16)
  p3 = _tc_mid(a2, p2, b28, W3, dis8)
  a3 = _make_agg(16)(p3, e3, z16)
  return _tc_last(a3, p3, b38, Wl, bl8, dis8)
